# Initial kernel scaffold; baseline (speedup 1.0000x reference)
import functools

import jax
import jax.numpy as jnp
from jax import lax
from jax.experimental import pallas as pl
from jax.experimental.pallas import tpu as pltpu
from jax.experimental.pallas import tpu_sc as plsc

N = 10000
E = 320000
HID = 160
EMB = 32
NGRAPH = 10
LPG = 1000

CHUNK = 128
NWORK = 32
CHUNKS_PER_W = (E + CHUNK * NWORK - 1) // (CHUNK * NWORK)
E_PAD = CHUNK * NWORK * CHUNKS_PER_W
ROWS_PER_TILE = N // 16


def _build_x_body(feat_ref, emb_ref, x_ref):
    f = feat_ref[0]
    op = f[:, 0].astype(jnp.int32)
    sel = (op[:, None] == lax.broadcasted_iota(jnp.int32, (1, 128), 1))
    emb = jnp.dot(sel.astype(jnp.float32), emb_ref[...],
                  preferred_element_type=jnp.float32)
    x_ref[...] = jnp.concatenate([emb, f[:, 1:]], axis=1)


def _build_x(features, emb_table):
    return pl.pallas_call(
        _build_x_body,
        grid=(N // LPG,),
        in_specs=[
            pl.BlockSpec((1, LPG, 129), lambda g: (0, g, 0)),
            pl.BlockSpec((128, EMB), lambda g: (0, 0)),
        ],
        out_specs=pl.BlockSpec((LPG, HID), lambda g: (g, 0)),
        out_shape=jax.ShapeDtypeStruct((N, HID), jnp.float32),
    )(features, emb_table)


_VROWS = 4096


def _val16_body(code_ref, val_ref):
    c = code_ref[...]
    v = jnp.exp2(-(c & 127).astype(jnp.float32))
    val_ref[...] = jnp.broadcast_to(v, (_VROWS, 16))


def _build_val16(codes2d):
    return pl.pallas_call(
        _val16_body,
        grid=(E_PAD // _VROWS,),
        in_specs=[pl.BlockSpec((_VROWS, 1), lambda g: (g, 0))],
        out_specs=pl.BlockSpec((_VROWS, 16), lambda g: (g, 0)),
        out_shape=jax.ShapeDtypeStruct((E_PAD, 16), jnp.float32),
    )(codes2d)


_MESH = plsc.VectorSubcoreMesh(core_axis_name="c", subcore_axis_name="s")


@functools.partial(
    pl.kernel,
    out_type=jax.ShapeDtypeStruct((2, N, HID), jnp.float32),
    mesh=_MESH,
    scratch_types=[
        pltpu.VMEM((CHUNK,), jnp.int32),
        pltpu.VMEM((CHUNK,), jnp.int32),
        pltpu.VMEM((CHUNK, 16), jnp.float32),
        pltpu.VMEM((CHUNK, HID), jnp.float32),
        pltpu.VMEM_SHARED((N, HID), jnp.float32),
        pltpu.SemaphoreType.DMA,
    ],
)
def _spmm(x_hbm, col_hbm, row_hbm, val_hbm, out_hbm,
          col_v, row_v, val_v, rows_v, acc_sh, sem):
    c = lax.axis_index("c")
    s = lax.axis_index("s")
    wid = c * 16 + s

    zero16 = jnp.zeros((16,), jnp.float32)

    def zrow(r, _):
        for j in range(HID // 16):
            rows_v[r, pl.ds(j * 16, 16)] = zero16
        return 0

    lax.fori_loop(0, 125, zrow, 0)
    for k in range(5):
        pltpu.sync_copy(rows_v.at[pl.ds(0, 125)],
                        acc_sh.at[pl.ds(s * ROWS_PER_TILE + k * 125, 125)])
    plsc.subcore_barrier()

    def chunk_body(i, _):
        off = (wid * CHUNKS_PER_W + i) * CHUNK
        pltpu.sync_copy(col_hbm.at[pl.ds(off, CHUNK)], col_v)
        pltpu.sync_copy(row_hbm.at[pl.ds(off, CHUNK)], row_v)
        pltpu.sync_copy(val_hbm.at[pl.ds(off, CHUNK)], val_v)
        pltpu.async_copy(x_hbm.at[col_v], rows_v, sem).wait()

        def edge_body(e, _):
            splat = val_v[e]
            for j in range(HID // 16):
                sl = pl.ds(j * 16, 16)
                rows_v[e, sl] = rows_v[e, sl] * splat
            return 0

        lax.fori_loop(0, CHUNK, edge_body, 0)
        pltpu.sync_copy(rows_v, acc_sh.at[row_v], add=True)
        return 0

    lax.fori_loop(0, CHUNKS_PER_W, chunk_body, 0)
    plsc.subcore_barrier()

    pltpu.sync_copy(acc_sh.at[pl.ds(s * ROWS_PER_TILE, ROWS_PER_TILE)],
                    out_hbm.at[c, pl.ds(s * ROWS_PER_TILE, ROWS_PER_TILE)])


def _layer_body(p_ref, w_ref, b_ref, o_ref):
    m = p_ref[0] + p_ref[1]
    h = jnp.dot(m, w_ref[...], preferred_element_type=jnp.float32)
    o_ref[...] = jnp.maximum(h + b_ref[...], 0.0)


def _layer(partials, W, b2d):
    return pl.pallas_call(
        _layer_body,
        grid=(N // LPG,),
        in_specs=[
            pl.BlockSpec((2, LPG, HID), lambda g: (0, g, 0)),
            pl.BlockSpec((HID, HID), lambda g: (0, 0)),
            pl.BlockSpec((1, HID), lambda g: (0, 0)),
        ],
        out_specs=pl.BlockSpec((LPG, HID), lambda g: (g, 0)),
        out_shape=jax.ShapeDtypeStruct((N, HID), jnp.float32),
    )(partials, W, b2d)


def _final_body(p_ref, w_ref, b_ref, wp_ref, bp_ref, o_ref):
    m = p_ref[0] + p_ref[1]
    h = jnp.dot(m, w_ref[...], preferred_element_type=jnp.float32)
    h = jnp.maximum(h + b_ref[...], 0.0)
    r = jnp.abs(jnp.dot(h, wp_ref[...],
                        preferred_element_type=jnp.float32) + bp_ref[...])
    o_ref[...] = jnp.full((8, 128), jnp.sum(r), jnp.float32)


def _final(partials, W, b2d, Wp, bp2d):
    return pl.pallas_call(
        _final_body,
        grid=(NGRAPH,),
        in_specs=[
            pl.BlockSpec((2, LPG, HID), lambda g: (0, g, 0)),
            pl.BlockSpec((HID, HID), lambda g: (0, 0)),
            pl.BlockSpec((1, HID), lambda g: (0, 0)),
            pl.BlockSpec((HID, 1), lambda g: (0, 0)),
            pl.BlockSpec((1, 1), lambda g: (0, 0)),
        ],
        out_specs=pl.BlockSpec((8, 128), lambda g: (g, 0)),
        out_shape=jax.ShapeDtypeStruct((8 * NGRAPH, 128), jnp.float32),
    )(partials, W, b2d, Wp, bp2d)


def kernel(features, row_indices, col_indices, edge_codes, lengths,
           emb_table, W1, b1, W2, b2, Wp, bp):
    pad = E_PAD - E
    col_p = jnp.concatenate([col_indices.astype(jnp.int32),
                             jnp.zeros((pad,), jnp.int32)])
    row_p = jnp.concatenate([row_indices.astype(jnp.int32),
                             jnp.zeros((pad,), jnp.int32)])
    codes2d = jnp.concatenate([edge_codes.astype(jnp.int32),
                               jnp.full((pad,), 127, jnp.int32)]).reshape(E_PAD, 1)

    x = _build_x(features, emb_table)
    val16 = _build_val16(codes2d)

    p1 = _spmm(x, col_p, row_p, val16)
    h1 = _layer(p1, W1, b1.reshape(1, HID))
    p2 = _spmm(h1, col_p, row_p, val16)
    outw = _final(p2, W2, b2.reshape(1, HID), Wp, bp.reshape(1, 1))
    return outw[::8, 0]


# speedup vs baseline: 7.0353x; 7.0353x over previous
import functools

import jax
import jax.numpy as jnp
from jax import lax
from jax.experimental import pallas as pl
from jax.experimental.pallas import tpu as pltpu
from jax.experimental.pallas import tpu_sc as plsc

N = 10000
E = 320000
HID = 160
HH = HID // 2
EMB = 32
NGRAPH = 10
LPG = 1000

CHUNK = 128
NBUF = 4
NCHUNKS = E // CHUNK
CHUNKS_LO = 156
CHUNKS_HI = 160
ACC_ROWS = 10240
ROWS_PER_TILE = ACC_ROWS // 16


def _build_x_body(feat_ref, emb_ref, x0_ref, x1_ref):
    f = feat_ref[0]
    op = f[:, 0].astype(jnp.int32)
    sel = (op[:, None] == lax.broadcasted_iota(jnp.int32, (1, 128), 1))
    emb = jnp.dot(sel.astype(jnp.float32), emb_ref[...],
                  preferred_element_type=jnp.float32)
    x = jnp.concatenate([emb, f[:, 1:]], axis=1)
    x0_ref[...] = x[:, :HH]
    x1_ref[...] = x[:, HH:]


def _build_x(features, emb_table):
    return pl.pallas_call(
        _build_x_body,
        grid=(N // LPG,),
        in_specs=[
            pl.BlockSpec((1, LPG, 129), lambda g: (0, g, 0)),
            pl.BlockSpec((128, EMB), lambda g: (0, 0)),
        ],
        out_specs=[pl.BlockSpec((LPG, HH), lambda g: (g, 0))] * 2,
        out_shape=[jax.ShapeDtypeStruct((N, HH), jnp.float32)] * 2,
    )(features, emb_table)


_VROWS = 4000


def _val16_body(code_ref, val_ref):
    c = code_ref[...]
    v = jnp.exp2(-(c & 127).astype(jnp.float32))
    val_ref[...] = jnp.broadcast_to(v, (_VROWS, 16))


def _build_val16(codes2d):
    return pl.pallas_call(
        _val16_body,
        grid=(E // _VROWS,),
        in_specs=[pl.BlockSpec((_VROWS, 1), lambda g: (g, 0))],
        out_specs=pl.BlockSpec((_VROWS, 16), lambda g: (g, 0)),
        out_shape=jax.ShapeDtypeStruct((E, 16), jnp.float32),
    )(codes2d)


_MESH = plsc.VectorSubcoreMesh(core_axis_name="c", subcore_axis_name="s")


@functools.partial(
    pl.kernel,
    out_type=jax.ShapeDtypeStruct((2, N, HH), jnp.float32),
    mesh=_MESH,
    compiler_params=pltpu.CompilerParams(use_tc_tiling_on_sc=False,
                                         internal_scratch_in_bytes=64 * 1024),
    scratch_types=[
        pltpu.VMEM((NBUF, 2, CHUNK), jnp.int32),
        pltpu.VMEM((NBUF, CHUNK, 16), jnp.float32),
        pltpu.VMEM((NBUF, CHUNK, HH), jnp.float32),
        pltpu.VMEM_SHARED((ACC_ROWS, HH), jnp.float32),
    ] + [pltpu.SemaphoreType.DMA] * (4 * NBUF),
)
def _spmm(x0_hbm, x1_hbm, idx_hbm, val_hbm, out_hbm,
          idx_v, val_v, rows_v, acc_sh, *sems):
    isem = sems[0:NBUF]
    gs, vs, ss = (sems[NBUF:2 * NBUF], sems[2 * NBUF:3 * NBUF],
                  sems[3 * NBUF:4 * NBUF])
    c = lax.axis_index("c")
    s = lax.axis_index("s")

    zero16 = jnp.zeros((16,), jnp.float32)

    def zrow(r, _):
        for j in range(HH // 16):
            rows_v[0, r, pl.ds(j * 16, 16)] = zero16
        return 0

    lax.fori_loop(0, CHUNK, zrow, 0)
    for k in range(ROWS_PER_TILE // CHUNK):
        pltpu.sync_copy(rows_v.at[0],
                        acc_sh.at[pl.ds(s * ROWS_PER_TILE + k * CHUNK, CHUNK)])

    n_chunks = jnp.where(s == 15, CHUNKS_HI, CHUNKS_LO)
    base = s * CHUNKS_LO

    def issue_idx(p, b):
        pltpu.async_copy(idx_hbm.at[base + p], idx_v.at[b], isem[b])
        pltpu.async_copy(val_hbm.at[pl.ds((base + p) * CHUNK, CHUNK)],
                         val_v.at[b], vs[b])

    def wait_idx(b):
        pltpu.make_async_copy(idx_hbm.at[0], idx_v.at[b], isem[b]).wait()

    def issue_gather(b):
        @pl.when(c == 0)
        def _():
            pltpu.async_copy(x0_hbm.at[idx_v.at[b, 0]], rows_v.at[b], gs[b])

        @pl.when(c == 1)
        def _():
            pltpu.async_copy(x1_hbm.at[idx_v.at[b, 0]], rows_v.at[b], gs[b])

    def wait_gather(b):
        pltpu.make_async_copy(x0_hbm.at[pl.ds(0, CHUNK)], rows_v.at[b],
                              gs[b]).wait()

    def wait_val(b):
        pltpu.make_async_copy(val_hbm.at[pl.ds(0, CHUNK)], val_v.at[b],
                              vs[b]).wait()

    def wait_scatter(b):
        pltpu.make_async_copy(x0_hbm.at[pl.ds(0, CHUNK)], rows_v.at[b],
                              ss[b]).wait()

    def scale(b):
        def grp(g, _):
            for l in range(16):
                e = g * 16 + l
                splat = val_v[b, e]
                for j in range(HH // 16):
                    sl = pl.ds(j * 16, 16)
                    rows_v[b, e, sl] = rows_v[b, e, sl] * splat
            return 0
        lax.fori_loop(0, CHUNK // 16, grp, 0)

    issue_idx(0, 0)
    issue_idx(1, 1)
    wait_idx(0)
    issue_gather(0)

    def chunk_group(i, _):
        for b in range(NBUF):
            ci = i * NBUF + b
            b2 = (b + 2) % NBUF
            b1 = (b + 1) % NBUF

            @pl.when(ci + 2 < n_chunks)
            def _():
                @pl.when(ci >= 2)
                def _():
                    wait_scatter(b2)

                issue_idx(ci + 2, b2)

            @pl.when(ci + 1 < n_chunks)
            def _():
                wait_idx(b1)
                issue_gather(b1)

            wait_gather(b)
            wait_val(b)
            scale(b)
            pltpu.async_copy(rows_v.at[b], acc_sh.at[idx_v.at[b, 1]],
                             ss[b], add=True)
        return 0

    lax.fori_loop(0, n_chunks // NBUF, chunk_group, 0)
    for b in range(NBUF):
        wait_scatter(b)
    plsc.subcore_barrier()

    @pl.when(s < 15)
    def _():
        pltpu.sync_copy(acc_sh.at[pl.ds(s * ROWS_PER_TILE, ROWS_PER_TILE)],
                        out_hbm.at[c, pl.ds(s * ROWS_PER_TILE, ROWS_PER_TILE)])

    @pl.when(s == 15)
    def _():
        pltpu.sync_copy(acc_sh.at[pl.ds(15 * ROWS_PER_TILE, N - 15 * ROWS_PER_TILE)],
                        out_hbm.at[c, pl.ds(15 * ROWS_PER_TILE, N - 15 * ROWS_PER_TILE)])


def _layer_body(p_ref, w_ref, b_ref, o0_ref, o1_ref):
    m = jnp.concatenate([p_ref[0], p_ref[1]], axis=1)
    h = jnp.dot(m, w_ref[...], preferred_element_type=jnp.float32)
    h = jnp.maximum(h + b_ref[...], 0.0)
    o0_ref[...] = h[:, :HH]
    o1_ref[...] = h[:, HH:]


def _layer(halves, W, b2d):
    return pl.pallas_call(
        _layer_body,
        grid=(N // LPG,),
        in_specs=[
            pl.BlockSpec((2, LPG, HH), lambda g: (0, g, 0)),
            pl.BlockSpec((HID, HID), lambda g: (0, 0)),
            pl.BlockSpec((1, HID), lambda g: (0, 0)),
        ],
        out_specs=[pl.BlockSpec((LPG, HH), lambda g: (g, 0))] * 2,
        out_shape=[jax.ShapeDtypeStruct((N, HH), jnp.float32)] * 2,
    )(halves, W, b2d)


def _final_body(h0_ref, h1_ref, wp_ref, bp_ref, o_ref):
    h = jnp.concatenate([h0_ref[...], h1_ref[...]], axis=1)
    r = jnp.abs(jnp.dot(h, wp_ref[...],
                        preferred_element_type=jnp.float32) + bp_ref[...])
    o_ref[...] = jnp.full((8, 128), jnp.sum(r), jnp.float32)


def _final(h0, h1, Wp, bp2d):
    return pl.pallas_call(
        _final_body,
        grid=(NGRAPH,),
        in_specs=[
            pl.BlockSpec((LPG, HH), lambda g: (g, 0)),
            pl.BlockSpec((LPG, HH), lambda g: (g, 0)),
            pl.BlockSpec((HID, 1), lambda g: (0, 0)),
            pl.BlockSpec((1, 1), lambda g: (0, 0)),
        ],
        out_specs=pl.BlockSpec((8, 128), lambda g: (g, 0)),
        out_shape=jax.ShapeDtypeStruct((8 * NGRAPH, 128), jnp.float32),
    )(h0, h1, Wp, bp2d)


def kernel(features, row_indices, col_indices, edge_codes, lengths,
           emb_table, W1, b1, W2, b2, Wp, bp):
    idx2 = jnp.stack([col_indices.astype(jnp.int32).reshape(NCHUNKS, CHUNK),
                      row_indices.astype(jnp.int32).reshape(NCHUNKS, CHUNK)],
                     axis=1)
    codes2d = edge_codes.astype(jnp.int32).reshape(E, 1)

    x0, x1 = _build_x(features, emb_table)
    val16 = _build_val16(codes2d)

    W_stack = jnp.stack([W1, W2])
    b_stack = jnp.stack([b1.reshape(1, HID), b2.reshape(1, HID)])

    def step(carry, wb):
        h0, h1 = carry
        W, b2d = wb
        p = _spmm(h0, h1, idx2, val16)
        n0, n1 = _layer(p, W, b2d)
        return (n0, n1), None

    (h0, h1), _ = lax.scan(step, (x0, x1), (W_stack, b_stack))
    outw = _final(h0, h1, Wp, bp.reshape(1, 1))
    return outw[::8, 0]


# speedup vs baseline: 14.0976x; 2.0038x over previous
import functools

import jax
import jax.numpy as jnp
from jax import lax
from jax.experimental import pallas as pl
from jax.experimental.pallas import tpu as pltpu
from jax.experimental.pallas import tpu_sc as plsc

N = 10000
E = 320000
HID = 160
HH = HID // 2
EMB = 32
NGRAPH = 10
LPG = 1000

CHUNK = 128
NBUF = 4
NCHUNKS = E // CHUNK
CHUNKS_LO = 156
CHUNKS_HI = 160
ACC_ROWS = 10240
ROWS_PER_TILE = ACC_ROWS // 16


def _build_x_body(feat_ref, emb_ref, x0_ref, x1_ref):
    f = feat_ref[0]
    op = f[:, 0].astype(jnp.int32)
    sel = (op[:, None] == lax.broadcasted_iota(jnp.int32, (1, 128), 1))
    emb = jnp.dot(sel.astype(jnp.float32), emb_ref[...],
                  preferred_element_type=jnp.float32)
    x = jnp.concatenate([emb, f[:, 1:]], axis=1)
    x0_ref[...] = x[:, :HH]
    x1_ref[...] = x[:, HH:]


def _build_x(features, emb_table):
    return pl.pallas_call(
        _build_x_body,
        grid=(N // LPG,),
        in_specs=[
            pl.BlockSpec((1, LPG, 129), lambda g: (0, g, 0)),
            pl.BlockSpec((128, EMB), lambda g: (0, 0)),
        ],
        out_specs=[pl.BlockSpec((LPG, HH), lambda g: (g, 0))] * 2,
        out_shape=[jax.ShapeDtypeStruct((N, HH), jnp.float32)] * 2,
    )(features, emb_table)


_VROWS = NCHUNKS


def _val_body(code_ref, val_ref):
    c = code_ref[...]
    val_ref[...] = jnp.exp2(-(c & 127).astype(jnp.float32))


def _build_val(codes):
    return pl.pallas_call(
        _val_body,
        grid=(NCHUNKS // _VROWS,),
        in_specs=[pl.BlockSpec((_VROWS, CHUNK), lambda g: (g, 0))],
        out_specs=pl.BlockSpec((_VROWS, CHUNK), lambda g: (g, 0)),
        out_shape=jax.ShapeDtypeStruct((NCHUNKS, CHUNK), jnp.float32),
    )(codes)


_MESH = plsc.VectorSubcoreMesh(core_axis_name="c", subcore_axis_name="s")


@functools.partial(
    pl.kernel,
    out_type=jax.ShapeDtypeStruct((2, N, HH), jnp.float32),
    mesh=_MESH,
    compiler_params=pltpu.CompilerParams(use_tc_tiling_on_sc=False,
                                         internal_scratch_in_bytes=64 * 1024),
    scratch_types=[
        pltpu.VMEM((NBUF, 2, CHUNK), jnp.int32),
        pltpu.VMEM((NBUF, CHUNK), jnp.float32),
        pltpu.VMEM((NBUF, CHUNK, HH), jnp.float32),
        pltpu.VMEM_SHARED((ACC_ROWS, HH), jnp.float32),
    ] + [pltpu.SemaphoreType.DMA] * (4 * NBUF),
)
def _spmm(x0_hbm, x1_hbm, idx_hbm, val_hbm, out_hbm,
          idx_v, val_v, rows_v, acc_sh, *sems):
    isem = sems[0:NBUF]
    gs, vs, ss = (sems[NBUF:2 * NBUF], sems[2 * NBUF:3 * NBUF],
                  sems[3 * NBUF:4 * NBUF])
    c = lax.axis_index("c")
    s = lax.axis_index("s")

    zero16 = jnp.zeros((16,), jnp.float32)

    def zrow(r, _):
        for j in range(HH // 16):
            rows_v[0, r, pl.ds(j * 16, 16)] = zero16
        return 0

    lax.fori_loop(0, CHUNK, zrow, 0)
    for k in range(ROWS_PER_TILE // CHUNK):
        pltpu.sync_copy(rows_v.at[0],
                        acc_sh.at[pl.ds(s * ROWS_PER_TILE + k * CHUNK, CHUNK)])

    n_chunks = jnp.where(s == 15, CHUNKS_HI, CHUNKS_LO)
    base = s * CHUNKS_LO

    def issue_idx(p, b):
        pltpu.async_copy(idx_hbm.at[base + p], idx_v.at[b], isem[b])
        pltpu.async_copy(val_hbm.at[base + p], val_v.at[b], vs[b])

    def wait_idx(b):
        pltpu.make_async_copy(idx_hbm.at[0], idx_v.at[b], isem[b]).wait()

    def issue_gather(b):
        @pl.when(c == 0)
        def _():
            pltpu.async_copy(x0_hbm.at[idx_v.at[b, 0]], rows_v.at[b], gs[b])

        @pl.when(c == 1)
        def _():
            pltpu.async_copy(x1_hbm.at[idx_v.at[b, 0]], rows_v.at[b], gs[b])

    def wait_gather(b):
        pltpu.make_async_copy(x0_hbm.at[pl.ds(0, CHUNK)], rows_v.at[b],
                              gs[b]).wait()

    def wait_val(b):
        pltpu.make_async_copy(val_hbm.at[0], val_v.at[b], vs[b]).wait()

    def wait_scatter(b):
        pltpu.make_async_copy(x0_hbm.at[pl.ds(0, CHUNK)], rows_v.at[b],
                              ss[b]).wait()

    def scale(b):
        def grp(g, _):
            vv = val_v[b, pl.ds(g * 16, 16)]
            for l in range(16):
                e = g * 16 + l
                splat = jnp.broadcast_to(vv[l], (16,))
                for j in range(HH // 16):
                    sl = pl.ds(j * 16, 16)
                    rows_v[b, e, sl] = rows_v[b, e, sl] * splat
            return 0
        lax.fori_loop(0, CHUNK // 16, grp, 0)

    issue_idx(0, 0)
    issue_idx(1, 1)
    wait_idx(0)
    issue_gather(0)

    def chunk_group(i, _):
        for b in range(NBUF):
            ci = i * NBUF + b
            b2 = (b + 2) % NBUF
            b1 = (b + 1) % NBUF

            @pl.when(ci + 2 < n_chunks)
            def _():
                @pl.when(ci >= 2)
                def _():
                    wait_scatter(b2)

                issue_idx(ci + 2, b2)

            @pl.when(ci + 1 < n_chunks)
            def _():
                wait_idx(b1)
                issue_gather(b1)

            wait_gather(b)
            wait_val(b)
            scale(b)
            pltpu.async_copy(rows_v.at[b], acc_sh.at[idx_v.at[b, 1]],
                             ss[b], add=True)
        return 0

    lax.fori_loop(0, n_chunks // NBUF, chunk_group, 0)
    for b in range(NBUF):
        wait_scatter(b)
    plsc.subcore_barrier()

    @pl.when(s < 15)
    def _():
        pltpu.sync_copy(acc_sh.at[pl.ds(s * ROWS_PER_TILE, ROWS_PER_TILE)],
                        out_hbm.at[c, pl.ds(s * ROWS_PER_TILE, ROWS_PER_TILE)])

    @pl.when(s == 15)
    def _():
        pltpu.sync_copy(acc_sh.at[pl.ds(15 * ROWS_PER_TILE, N - 15 * ROWS_PER_TILE)],
                        out_hbm.at[c, pl.ds(15 * ROWS_PER_TILE, N - 15 * ROWS_PER_TILE)])


def _layer_body(p_ref, w_ref, b_ref, o0_ref, o1_ref):
    m = jnp.concatenate([p_ref[0], p_ref[1]], axis=1)
    h = jnp.dot(m, w_ref[...], preferred_element_type=jnp.float32)
    h = jnp.maximum(h + b_ref[...], 0.0)
    o0_ref[...] = h[:, :HH]
    o1_ref[...] = h[:, HH:]


def _layer(halves, W, b2d):
    return pl.pallas_call(
        _layer_body,
        grid=(N // LPG,),
        in_specs=[
            pl.BlockSpec((2, LPG, HH), lambda g: (0, g, 0)),
            pl.BlockSpec((HID, HID), lambda g: (0, 0)),
            pl.BlockSpec((1, HID), lambda g: (0, 0)),
        ],
        out_specs=[pl.BlockSpec((LPG, HH), lambda g: (g, 0))] * 2,
        out_shape=[jax.ShapeDtypeStruct((N, HH), jnp.float32)] * 2,
    )(halves, W, b2d)


def _final_body(h0_ref, h1_ref, wp_ref, bp_ref, o_ref):
    h = jnp.concatenate([h0_ref[...], h1_ref[...]], axis=1)
    r = jnp.abs(jnp.dot(h, wp_ref[...],
                        preferred_element_type=jnp.float32) + bp_ref[...])
    o_ref[...] = jnp.full((8, 128), jnp.sum(r), jnp.float32)


def _final(h0, h1, Wp, bp2d):
    return pl.pallas_call(
        _final_body,
        grid=(NGRAPH,),
        in_specs=[
            pl.BlockSpec((LPG, HH), lambda g: (g, 0)),
            pl.BlockSpec((LPG, HH), lambda g: (g, 0)),
            pl.BlockSpec((HID, 1), lambda g: (0, 0)),
            pl.BlockSpec((1, 1), lambda g: (0, 0)),
        ],
        out_specs=pl.BlockSpec((8, 128), lambda g: (g, 0)),
        out_shape=jax.ShapeDtypeStruct((8 * NGRAPH, 128), jnp.float32),
    )(h0, h1, Wp, bp2d)


def kernel(features, row_indices, col_indices, edge_codes, lengths,
           emb_table, W1, b1, W2, b2, Wp, bp):
    idx2 = jnp.stack([col_indices.astype(jnp.int32).reshape(NCHUNKS, CHUNK),
                      row_indices.astype(jnp.int32).reshape(NCHUNKS, CHUNK)],
                     axis=1)
    codes = edge_codes.astype(jnp.int32).reshape(NCHUNKS, CHUNK)

    x0, x1 = _build_x(features, emb_table)
    vals = _build_val(codes)

    W_stack = jnp.stack([W1, W2])
    b_stack = jnp.stack([b1.reshape(1, HID), b2.reshape(1, HID)])

    def step(carry, wb):
        h0, h1 = carry
        W, b2d = wb
        p = _spmm(h0, h1, idx2, vals)
        n0, n1 = _layer(p, W, b2d)
        return (n0, n1), None

    (h0, h1), _ = lax.scan(step, (x0, x1), (W_stack, b_stack))
    outw = _final(h0, h1, Wp, bp.reshape(1, 1))
    return outw[::8, 0]


# speedup vs baseline: 14.2323x; 1.0096x over previous
import functools

import jax
import jax.numpy as jnp
from jax import lax
from jax.experimental import pallas as pl
from jax.experimental.pallas import tpu as pltpu
from jax.experimental.pallas import tpu_sc as plsc

N = 10000
E = 320000
HID = 160
HH = HID // 2
EMB = 32
NGRAPH = 10
LPG = 1000

CHUNK = 128
NBUF = 4
NCHUNKS = E // CHUNK
CHUNKS_LO = 156
CHUNKS_HI = 160
ACC_ROWS = 10240
ROWS_PER_TILE = ACC_ROWS // 16


def _build_x_body(feat_ref, emb_ref, x0_ref, x1_ref):
    f = feat_ref[0]
    op = f[:, 0].astype(jnp.int32)
    sel = (op[:, None] == lax.broadcasted_iota(jnp.int32, (1, 128), 1))
    emb = jnp.dot(sel.astype(jnp.float32), emb_ref[...],
                  preferred_element_type=jnp.float32)
    x = jnp.concatenate([emb, f[:, 1:]], axis=1)
    x0_ref[...] = x[:, :HH]
    x1_ref[...] = x[:, HH:]


def _build_x(features, emb_table):
    return pl.pallas_call(
        _build_x_body,
        grid=(N // LPG,),
        in_specs=[
            pl.BlockSpec((1, LPG, 129), lambda g: (0, g, 0)),
            pl.BlockSpec((128, EMB), lambda g: (0, 0)),
        ],
        out_specs=[pl.BlockSpec((LPG, HH), lambda g: (g, 0))] * 2,
        out_shape=[jax.ShapeDtypeStruct((N, HH), jnp.float32)] * 2,
    )(features, emb_table)


_VROWS = NCHUNKS


def _val_body(code_ref, val_ref):
    c = code_ref[...]
    val_ref[...] = jnp.exp2(-(c & 127).astype(jnp.float32))


def _build_val(codes):
    return pl.pallas_call(
        _val_body,
        grid=(NCHUNKS // _VROWS,),
        in_specs=[pl.BlockSpec((_VROWS, CHUNK), lambda g: (g, 0))],
        out_specs=pl.BlockSpec((_VROWS, CHUNK), lambda g: (g, 0)),
        out_shape=jax.ShapeDtypeStruct((NCHUNKS, CHUNK), jnp.float32),
    )(codes)


_MESH = plsc.VectorSubcoreMesh(core_axis_name="c", subcore_axis_name="s")


@functools.partial(
    pl.kernel,
    out_type=jax.ShapeDtypeStruct((2, N, HH), jnp.float32),
    mesh=_MESH,
    compiler_params=pltpu.CompilerParams(use_tc_tiling_on_sc=False,
                                         internal_scratch_in_bytes=64 * 1024),
    scratch_types=[
        pltpu.VMEM((NBUF, 2, CHUNK), jnp.int32),
        pltpu.VMEM((NBUF, CHUNK), jnp.float32),
        pltpu.VMEM((NBUF, CHUNK, HH), jnp.float32),
        pltpu.VMEM_SHARED((ACC_ROWS, HH), jnp.float32),
    ] + [pltpu.SemaphoreType.DMA] * (4 * NBUF),
)
def _spmm(x0_hbm, x1_hbm, idx_hbm, val_hbm, out_hbm,
          idx_v, val_v, rows_v, acc_sh, *sems):
    isem = sems[0:NBUF]
    gs, vs, ss = (sems[NBUF:2 * NBUF], sems[2 * NBUF:3 * NBUF],
                  sems[3 * NBUF:4 * NBUF])
    c = lax.axis_index("c")
    s = lax.axis_index("s")

    zero16 = jnp.zeros((16,), jnp.float32)

    def zrow(r, _):
        for j in range(HH // 16):
            rows_v[0, r, pl.ds(j * 16, 16)] = zero16
        return 0

    lax.fori_loop(0, CHUNK, zrow, 0)
    for k in range(ROWS_PER_TILE // CHUNK):
        pltpu.sync_copy(rows_v.at[0],
                        acc_sh.at[pl.ds(s * ROWS_PER_TILE + k * CHUNK, CHUNK)])

    n_chunks = jnp.where(s == 15, CHUNKS_HI, CHUNKS_LO)
    base = s * CHUNKS_LO

    def issue_idx(p, b):
        pltpu.async_copy(idx_hbm.at[base + p], idx_v.at[b], isem[b])
        pltpu.async_copy(val_hbm.at[base + p], val_v.at[b], vs[b])

    def wait_idx(b):
        pltpu.make_async_copy(idx_hbm.at[0], idx_v.at[b], isem[b]).wait()

    def issue_gather(b):
        @pl.when(c == 0)
        def _():
            pltpu.async_copy(x0_hbm.at[idx_v.at[b, 0]], rows_v.at[b], gs[b])

        @pl.when(c == 1)
        def _():
            pltpu.async_copy(x1_hbm.at[idx_v.at[b, 0]], rows_v.at[b], gs[b])

    def wait_gather(b):
        pltpu.make_async_copy(x0_hbm.at[pl.ds(0, CHUNK)], rows_v.at[b],
                              gs[b]).wait()

    def wait_val(b):
        pltpu.make_async_copy(val_hbm.at[0], val_v.at[b], vs[b]).wait()

    def wait_scatter(b):
        pltpu.make_async_copy(x0_hbm.at[pl.ds(0, CHUNK)], rows_v.at[b],
                              ss[b]).wait()

    def scale(b):
        def grp(g, _):
            vv = val_v[b, pl.ds(g * 16, 16)]
            for l in range(16):
                e = g * 16 + l
                splat = jnp.broadcast_to(vv[l], (16,))
                for j in range(HH // 16):
                    sl = pl.ds(j * 16, 16)
                    rows_v[b, e, sl] = rows_v[b, e, sl] * splat
            return 0
        lax.fori_loop(0, CHUNK // 16, grp, 0)

    issue_idx(0, 0)
    issue_idx(1, 1)
    wait_idx(0)
    issue_gather(0)

    def chunk_group(i, _):
        for b in range(NBUF):
            ci = i * NBUF + b
            b2 = (b + 2) % NBUF
            b1 = (b + 1) % NBUF

            @pl.when(ci + 2 < n_chunks)
            def _():
                @pl.when(ci >= 2)
                def _():
                    wait_scatter(b2)

                issue_idx(ci + 2, b2)

            @pl.when(ci + 1 < n_chunks)
            def _():
                wait_idx(b1)
                issue_gather(b1)

            wait_gather(b)
            wait_val(b)
            scale(b)
            pltpu.async_copy(rows_v.at[b], acc_sh.at[idx_v.at[b, 1]],
                             ss[b], add=True)
        return 0

    lax.fori_loop(0, n_chunks // NBUF, chunk_group, 0)
    for b in range(NBUF):
        wait_scatter(b)
    plsc.subcore_barrier()

    @pl.when(s < 15)
    def _():
        pltpu.sync_copy(acc_sh.at[pl.ds(s * ROWS_PER_TILE, ROWS_PER_TILE)],
                        out_hbm.at[c, pl.ds(s * ROWS_PER_TILE, ROWS_PER_TILE)])

    @pl.when(s == 15)
    def _():
        pltpu.sync_copy(acc_sh.at[pl.ds(15 * ROWS_PER_TILE, N - 15 * ROWS_PER_TILE)],
                        out_hbm.at[c, pl.ds(15 * ROWS_PER_TILE, N - 15 * ROWS_PER_TILE)])


def _layer_body(p_ref, w_ref, b_ref, o0_ref, o1_ref):
    m = jnp.concatenate([p_ref[0], p_ref[1]], axis=1)
    h = jnp.dot(m, w_ref[...], preferred_element_type=jnp.float32)
    h = jnp.maximum(h + b_ref[...], 0.0)
    o0_ref[...] = h[:, :HH]
    o1_ref[...] = h[:, HH:]


def _layer(halves, W, b2d):
    return pl.pallas_call(
        _layer_body,
        grid=(N // LPG,),
        in_specs=[
            pl.BlockSpec((2, LPG, HH), lambda g: (0, g, 0)),
            pl.BlockSpec((HID, HID), lambda g: (0, 0)),
            pl.BlockSpec((1, HID), lambda g: (0, 0)),
        ],
        out_specs=[pl.BlockSpec((LPG, HH), lambda g: (g, 0))] * 2,
        out_shape=[jax.ShapeDtypeStruct((N, HH), jnp.float32)] * 2,
    )(halves, W, b2d)


def _final_body(h0_ref, h1_ref, wp_ref, bp_ref, o_ref):
    h = jnp.concatenate([h0_ref[...], h1_ref[...]], axis=1)
    r = jnp.abs(jnp.dot(h, wp_ref[...],
                        preferred_element_type=jnp.float32) + bp_ref[...])
    o_ref[...] = jnp.full((8, 128), jnp.sum(r), jnp.float32)


def _final(h0, h1, Wp, bp2d):
    return pl.pallas_call(
        _final_body,
        grid=(NGRAPH,),
        in_specs=[
            pl.BlockSpec((LPG, HH), lambda g: (g, 0)),
            pl.BlockSpec((LPG, HH), lambda g: (g, 0)),
            pl.BlockSpec((HID, 1), lambda g: (0, 0)),
            pl.BlockSpec((1, 1), lambda g: (0, 0)),
        ],
        out_specs=pl.BlockSpec((8, 128), lambda g: (g, 0)),
        out_shape=jax.ShapeDtypeStruct((8 * NGRAPH, 128), jnp.float32),
    )(h0, h1, Wp, bp2d)


def kernel(features, row_indices, col_indices, edge_codes, lengths,
           emb_table, W1, b1, W2, b2, Wp, bp):
    idx2 = jnp.stack([col_indices.astype(jnp.int32).reshape(NCHUNKS, CHUNK),
                      row_indices.astype(jnp.int32).reshape(NCHUNKS, CHUNK)],
                     axis=1)
    codes = edge_codes.astype(jnp.int32).reshape(NCHUNKS, CHUNK)

    x0, x1 = _build_x(features, emb_table)
    vals = _build_val(codes)

    p1 = _spmm(x0, x1, idx2, vals)
    h10, h11 = _layer(p1, W1, b1.reshape(1, HID))
    p2 = _spmm(h10, h11, idx2, vals)
    h20, h21 = _layer(p2, W2, b2.reshape(1, HID))
    outw = _final(h20, h21, Wp, bp.reshape(1, 1))
    return outw[::8, 0]
